# baseline (speedup 1.0000x reference)
import functools

import jax
import jax.numpy as jnp
from jax import lax
from jax.experimental import pallas as pl
from jax.experimental.pallas import tpu as pltpu
from jax.experimental.pallas import tpu_sc as plsc

N = 10000
NPAD = 10240
E = 320000
K = 128
NW = 32
NCH = 80
EPAD = NW * NCH * K
ACCROWS = 10016
RPS = ACCROWS // 16
TB = 4
NBUF = 2
ROUNDS = NCH // TB


@functools.lru_cache(maxsize=None)
def _sc_agg():
    mesh = plsc.VectorSubcoreMesh(
        core_axis_name="c", subcore_axis_name="s", num_cores=2, num_subcores=16)

    def body(xwa_hbm, xwb_hbm, src_hbm, dst_hbm, zero_hbm, np_hbm, out_hbm,
             sidx, didx, rows, npv, acc, *sems):
        gsems, ssems = sems[:NBUF], sems[NBUF:]
        c = lax.axis_index("c")
        s = lax.axis_index("s")
        wid = s * 2 + c
        pltpu.sync_copy(src_hbm.at[wid], sidx)
        pltpu.sync_copy(dst_hbm.at[wid], didx)
        pltpu.sync_copy(np_hbm, npv)
        nparts = npv[pl.ds(0, 16)][0]

        def phase(xw_hbm, p):
            pltpu.sync_copy(zero_hbm.at[pl.ds(s * RPS, RPS)],
                            acc.at[pl.ds(s * RPS, RPS)])
            for b in range(NBUF):
                pltpu.async_copy(xw_hbm.at[sidx.at[b]], rows.at[b], gsems[b])
            plsc.subcore_barrier()

            def half(g, bank):
                pltpu.make_async_copy(
                    xw_hbm.at[sidx.at[g]], rows.at[bank],
                    gsems[bank]).wait()
                pltpu.async_copy(
                    rows.at[bank], acc.at[didx.at[g]], ssems[bank],
                    add=True).wait()

                @pl.when(g + 2 < ROUNDS)
                def _():
                    pltpu.async_copy(xw_hbm.at[sidx.at[g + 2]],
                                     rows.at[bank], gsems[bank])

            def pair(j, carry):
                half(2 * j, 0)
                half(2 * j + 1, 1)
                return carry

            lax.fori_loop(0, ROUNDS // 2, pair, 0)
            plsc.subcore_barrier()
            pltpu.sync_copy(acc.at[pl.ds(s * RPS, RPS)],
                            out_hbm.at[p, c, pl.ds(s * RPS, RPS)])

        phase(xwa_hbm, 0)

        @pl.when(nparts > 1)
        def _():
            plsc.subcore_barrier()
            phase(xwb_hbm, 1)

    return pl.kernel(
        body,
        out_type=jax.ShapeDtypeStruct((2, 2, ACCROWS, 64), jnp.float32),
        mesh=mesh,
        compiler_params=pltpu.CompilerParams(use_tc_tiling_on_sc=False),
        scratch_types=[
            pltpu.VMEM((ROUNDS, TB * K), jnp.int32),
            pltpu.VMEM((ROUNDS, TB * K), jnp.int32),
            pltpu.VMEM((NBUF, TB * K, 64), jnp.float32),
            pltpu.VMEM((16,), jnp.int32),
            pltpu.VMEM_SHARED((ACCROWS, 64), jnp.float32),
        ] + [pltpu.SemaphoreType.DMA] * (2 * NBUF),
    )


_BM = 1024


def _mm_scale(x, W, b, degp):
    din, dout = W.shape
    nch = dout // 64

    def body(x_ref, w_ref, b_ref, deg_ref, *out_refs):
        xs_refs, base_ref = out_refs[:nch], out_refs[nch]
        xw = jnp.dot(x_ref[...], w_ref[...], preferred_element_type=jnp.float32)
        deg = deg_ref[0, :, 0:1] + deg_ref[1, :, 0:1] + 1.0
        xs = xw * lax.rsqrt(deg)
        for c in range(nch):
            xs_refs[c][...] = xs[:, c * 64:(c + 1) * 64]
        base_ref[...] = xw * (1.0 / deg) + b_ref[...]

    outs = pl.pallas_call(
        body,
        grid=(NPAD // _BM,),
        in_specs=[
            pl.BlockSpec((_BM, din), lambda i: (i, 0)),
            pl.BlockSpec((din, dout), lambda i: (0, 0)),
            pl.BlockSpec((1, dout), lambda i: (0, 0)),
            pl.BlockSpec((2, _BM, 64), lambda i: (0, i, 0)),
        ],
        out_specs=[pl.BlockSpec((_BM, 64), lambda i: (i, 0))] * nch
        + [pl.BlockSpec((_BM, dout), lambda i: (i, 0))],
        out_shape=[jax.ShapeDtypeStruct((NPAD, 64), jnp.float32)] * nch
        + [jax.ShapeDtypeStruct((NPAD, dout), jnp.float32)],
    )(x, W, b.reshape(1, dout), degp)
    return outs[:nch], outs[nch]


def _combine(aggs, base, degp, relu):
    d = base.shape[1]
    nch = len(aggs)

    def body(*refs):
        agg_refs, (base_ref, deg_ref, out_ref) = refs[:nch], refs[nch:]
        acc = jnp.concatenate([r[0] + r[1] for r in agg_refs], axis=1)
        deg = deg_ref[0, :, 0:1] + deg_ref[1, :, 0:1] + 1.0
        out = acc * lax.rsqrt(deg) + base_ref[...]
        out_ref[...] = jnp.maximum(out, 0.0) if relu else out

    return pl.pallas_call(
        body,
        grid=(NPAD // _BM,),
        in_specs=[pl.BlockSpec((2, _BM, 64), lambda i: (0, i, 0))] * nch
        + [
            pl.BlockSpec((_BM, d), lambda i: (i, 0)),
            pl.BlockSpec((2, _BM, 64), lambda i: (0, i, 0)),
        ],
        out_specs=pl.BlockSpec((_BM, d), lambda i: (i, 0)),
        out_shape=jax.ShapeDtypeStruct((NPAD, d), jnp.float32),
    )(*aggs, base, degp)


def _selfmm(s):
    BM, BN = 1000, 1280

    def body(a_ref, b_ref, o_ref):
        o_ref[...] = lax.dot_general(
            a_ref[...], b_ref[...], (((1,), (1,)), ((), ())),
            preferred_element_type=jnp.float32)

    return pl.pallas_call(
        body,
        grid=(N // BM, NPAD // BN),
        in_specs=[pl.BlockSpec((BM, 64), lambda i, j: (i, 0)),
                  pl.BlockSpec((BN, 64), lambda i, j: (j, 0))],
        out_specs=pl.BlockSpec((BM, BN), lambda i, j: (i, j)),
        out_shape=jax.ShapeDtypeStruct((N, N), jnp.float32),
    )(s, s)


def kernel(x, edge_index, W1, b1, W2, b2, W3, b3, W4, b4, W5, b5):
    npd = EPAD - E
    srcp = jnp.concatenate(
        [edge_index[0], jnp.zeros((npd,), jnp.int32)]).reshape(
            NW, ROUNDS, TB * K)
    dstp = jnp.concatenate(
        [edge_index[1], jnp.full((npd,), N, jnp.int32)]).reshape(
            NW, ROUNDS, TB * K)

    xpad = jnp.concatenate([x, jnp.zeros((NPAD - N, x.shape[1]), jnp.float32)])
    ones64 = jnp.ones((NPAD, 64), jnp.float32)
    z64 = jnp.zeros((ACCROWS, 64), jnp.float32)
    np1 = jnp.full((16,), 1, jnp.int32)
    np2 = jnp.full((16,), 2, jnp.int32)
    aggm = _sc_agg()

    degp = aggm(ones64, ones64, srcp, dstp, z64, np1)[0]

    def conv(xin, W, b, relu):
        xs_parts, base = _mm_scale(xin, W, b, degp)
        if len(xs_parts) == 2:
            out = aggm(xs_parts[0], xs_parts[1], srcp, dstp, z64, np2)
            aggs = [out[0], out[1]]
        else:
            out = aggm(xs_parts[0], xs_parts[0], srcp, dstp, z64, np1)
            aggs = [out[0]]
        return _combine(aggs, base, degp, relu)

    h = conv(xpad, W1, b1, True)
    z = conv(h, W2, b2, True)
    a = conv(z, W3, b3, True)
    x_rec = conv(a, W4, b4, False)[:N]
    s = conv(z, W5, b5, True)
    adj_rec = _selfmm(s)
    return (x_rec, adj_rec)
